# baseline (speedup 1.0000x reference)
import functools

import jax
import jax.numpy as jnp
from jax import lax
from jax.experimental import pallas as pl
from jax.experimental.pallas import tpu as pltpu
from jax.experimental.pallas import tpu_sc as plsc


def _sc_gather(table, idx, D):
    V = table.shape[0]
    (B,) = idx.shape
    info = plsc.get_sparse_core_info()
    NC, NS = info.num_cores, info.num_subcores
    NW = NC * NS
    assert B % NW == 0
    b_per_w = B // NW
    mesh = plsc.VectorSubcoreMesh(core_axis_name="c", subcore_axis_name="s")

    @functools.partial(
        pl.kernel,
        mesh=mesh,
        out_type=jax.ShapeDtypeStruct((B, D), jnp.float32),
        scratch_types=[
            pltpu.VMEM((b_per_w,), jnp.int32),
            pltpu.VMEM((b_per_w, D), jnp.float32),
            pltpu.SemaphoreType.DMA,
        ],
    )
    def gather_kernel(table_hbm, idx_hbm, out_hbm, idx_v, rows_v, sem):
        wid = lax.axis_index("s") * NC + lax.axis_index("c")
        base = wid * b_per_w
        pltpu.sync_copy(idx_hbm.at[pl.ds(base, b_per_w)], idx_v)
        pltpu.async_copy(table_hbm.at[idx_v], rows_v, sem).wait()
        pltpu.sync_copy(rows_v, out_hbm.at[pl.ds(base, b_per_w)])

    return gather_kernel(table, idx)


def _tc_project(x2, par, Wt, b2d):
    B, D2 = x2.shape
    D = D2 // 2
    N = Wt.shape[1]
    BM = 1024
    assert B % BM == 0

    def body(x_ref, p_ref, w_ref, b_ref, o_ref):
        x = x_ref[...]
        xsel = jnp.where(p_ref[...] > 0.5, x[:, D:], x[:, :D])
        o_ref[...] = (
            jnp.dot(xsel, w_ref[...], preferred_element_type=jnp.float32)
            + b_ref[...]
        )

    return pl.pallas_call(
        body,
        grid=(B // BM,),
        in_specs=[
            pl.BlockSpec((BM, D2), lambda i: (i, 0)),
            pl.BlockSpec((BM, 1), lambda i: (i, 0)),
            pl.BlockSpec((D, N), lambda i: (0, 0)),
            pl.BlockSpec((1, N), lambda i: (0, 0)),
        ],
        out_specs=pl.BlockSpec((BM, N), lambda i: (i, 0)),
        out_shape=jax.ShapeDtypeStruct((B, N), jnp.float32),
    )(x2, par, Wt, b2d)


def kernel(doc_id, doc_emb, W, b):
    V, D = doc_emb.shape
    idx = doc_id.astype(jnp.int32)
    pair_idx = idx >> 1
    parity = (idx & 1).astype(jnp.float32).reshape(-1, 1)
    table2 = doc_emb.reshape(V // 2, 2 * D)
    doc_pair = _sc_gather(table2, pair_idx, 2 * D)
    return _tc_project(doc_pair, parity, W.T, b.reshape(1, -1))


# speedup vs baseline: 1.0804x; 1.0804x over previous
import functools

import jax
import jax.numpy as jnp
from jax import lax
from jax.experimental import pallas as pl
from jax.experimental.pallas import tpu as pltpu
from jax.experimental.pallas import tpu_sc as plsc


def _sc_gather(table, idx, D):
    V = table.shape[0]
    (B,) = idx.shape
    info = plsc.get_sparse_core_info()
    NC, NS = info.num_cores, info.num_subcores
    NW = NC * NS
    assert B % NW == 0
    b_per_w = B // NW
    mesh = plsc.VectorSubcoreMesh(core_axis_name="c", subcore_axis_name="s")

    @functools.partial(
        pl.kernel,
        mesh=mesh,
        out_type=jax.ShapeDtypeStruct((B, D), jnp.float32),
        scratch_types=[
            pltpu.VMEM((b_per_w,), jnp.int32),
            pltpu.VMEM((b_per_w, D), jnp.float32),
            pltpu.SemaphoreType.DMA,
        ],
    )
    def gather_kernel(table_hbm, idx_hbm, out_hbm, idx_v, rows_v, sem):
        wid = lax.axis_index("s") * NC + lax.axis_index("c")
        base = wid * b_per_w
        pltpu.sync_copy(idx_hbm.at[pl.ds(base, b_per_w)], idx_v)
        pltpu.async_copy(table_hbm.at[idx_v], rows_v, sem).wait()
        pltpu.sync_copy(rows_v, out_hbm.at[pl.ds(base, b_per_w)])

    return gather_kernel(table, idx)


def _tc_project_t(x2, par, W, b2d):
    B, D2 = x2.shape
    D = D2 // 2
    N = W.shape[0]
    BM = 1024
    assert B % BM == 0

    def body(x_ref, p_ref, w_ref, b_ref, o_ref):
        x = x_ref[...]
        xsel = jnp.where(p_ref[...] > 0.5, x[:, D:], x[:, :D])
        o_ref[...] = (
            lax.dot_general(
                w_ref[...],
                xsel,
                (((1,), (1,)), ((), ())),
                preferred_element_type=jnp.float32,
            )
            + b_ref[...]
        )

    return pl.pallas_call(
        body,
        grid=(B // BM,),
        in_specs=[
            pl.BlockSpec((BM, D2), lambda i: (i, 0)),
            pl.BlockSpec((BM, 1), lambda i: (i, 0)),
            pl.BlockSpec((N, D), lambda i: (0, 0)),
            pl.BlockSpec((N, 1), lambda i: (0, 0)),
        ],
        out_specs=pl.BlockSpec((N, BM), lambda i: (0, i)),
        out_shape=jax.ShapeDtypeStruct((N, B), jnp.float32),
    )(x2, par, W, b2d)


def kernel(doc_id, doc_emb, W, b):
    V, D = doc_emb.shape
    idx = doc_id.astype(jnp.int32)
    pair_idx = idx >> 1
    parity = (idx & 1).astype(jnp.float32).reshape(-1, 1)
    table2 = doc_emb.reshape(V // 2, 2 * D)
    doc_pair = _sc_gather(table2, pair_idx, 2 * D)
    logits_t = _tc_project_t(doc_pair, parity, W, b.reshape(-1, 1))
    return logits_t.T
